# Initial kernel scaffold; baseline (speedup 1.0000x reference)
import jax
import jax.numpy as jnp
from jax.experimental import pallas as pl


def kernel(x, edge_index, edge_weight):
    raise NotImplementedError("write your pallas kernel here")


# speedup vs baseline: 2.4833x; 2.4833x over previous
import dataclasses
import functools

import jax
import jax.numpy as jnp
from jax import lax
from jax.experimental import pallas as pl
from jax.experimental.pallas import tpu as pltpu
from jax.experimental.pallas import tpu_sc as plsc

_EPS = 1e-15
_NUM_LAYERS = 3
_NC = 2
_NS = 16
_L = 16
_CH = 128


def _pre_logmap(x, hd):
    n, d = x.shape
    blk = 1000

    def body(x_ref, o_ref):
        xb = x_ref[...]
        pn = jnp.sqrt(jnp.sum(xb * xb, axis=1, keepdims=True))
        pn = jnp.maximum(pn, _EPS)
        z = jnp.clip(pn, -1.0 + 1e-7, 1.0 - 1e-7)
        at = 0.5 * (jnp.log1p(z) - jnp.log1p(-z))
        xt = xb / pn * at
        o_ref[0] = xt[:, :hd]
        o_ref[1] = xt[:, hd:]

    out = pl.pallas_call(
        body,
        grid=(n // blk,),
        in_specs=[pl.BlockSpec((blk, d), lambda i: (i, 0))],
        out_specs=pl.BlockSpec((2, blk, hd), lambda i: (0, i, 0)),
        out_shape=jax.ShapeDtypeStruct((2, n, hd), jnp.float32),
    )(x)
    return out.reshape(2 * n, hd)


def _spmm_sc(src, dst, w, xt, n, hd):
    e_pad = src.shape[0]
    ept = e_pad // _NS
    nch = ept // _CH
    rpt = (n // _NS) // 8 * 8
    tail = n - _NS * rpt
    full = rpt // _CH
    rem = rpt - full * _CH

    mesh = plsc.VectorSubcoreMesh(core_axis_name="c", subcore_axis_name="s")
    cp = pltpu.CompilerParams()
    if "needs_layout_passes" in pltpu.CompilerParams.__dataclass_fields__:
        cp = dataclasses.replace(cp, needs_layout_passes=False)

    @functools.partial(
        pl.kernel,
        out_type=jax.ShapeDtypeStruct((2 * n, hd), jnp.float32),
        mesh=mesh,
        compiler_params=cp,
        scratch_types=[
            pltpu.VMEM_SHARED((n, hd), jnp.float32),
            pltpu.VMEM((_CH,), jnp.int32),
            pltpu.VMEM((_CH,), jnp.int32),
            pltpu.VMEM((_CH,), jnp.float32),
            pltpu.VMEM((_CH, hd), jnp.float32),
        ],
    )
    def run(src_hbm, dst_hbm, w_hbm, x_hbm, out_hbm, acc, sidx, didx, wbuf, rows):
        c = lax.axis_index("c")
        s = lax.axis_index("s")
        coff = c * n
        base = s * rpt

        @pl.loop(0, _CH)
        def _(i):
            for j in range(hd // _L):
                rows[i, pl.ds(j * _L, _L)] = jnp.zeros((_L,), jnp.float32)

        for k in range(full):
            pltpu.sync_copy(rows, acc.at[pl.ds(base + k * _CH, _CH)])
        if rem:
            pltpu.sync_copy(rows.at[pl.ds(0, rem)],
                            acc.at[pl.ds(base + full * _CH, rem)])
        if tail:
            @pl.when(s == _NS - 1)
            def _():
                pltpu.sync_copy(rows.at[pl.ds(0, tail)],
                                acc.at[pl.ds(_NS * rpt, tail)])
        plsc.subcore_barrier()

        e_base = s * ept

        @pl.loop(0, nch)
        def _(k):
            e0 = e_base + k * _CH
            pltpu.sync_copy(src_hbm.at[pl.ds(e0, _CH)], sidx)
            pltpu.sync_copy(dst_hbm.at[pl.ds(e0, _CH)], didx)
            pltpu.sync_copy(w_hbm.at[pl.ds(e0, _CH)], wbuf)

            @pl.loop(0, _CH, step=_L)
            def _(i):
                sidx[pl.ds(i, _L)] = sidx[pl.ds(i, _L)] + coff

            pltpu.sync_copy(x_hbm.at[sidx], rows)

            @pl.loop(0, _CH)
            def _(e):
                wv = plsc.load_gather(wbuf, [jnp.full((_L,), e, jnp.int32)])
                for j in range(hd // _L):
                    sl = pl.ds(j * _L, _L)
                    rows[e, sl] = rows[e, sl] * wv

            pltpu.sync_copy(rows, acc.at[didx], add=True)

        plsc.subcore_barrier()

        obase = coff + base
        for k in range(full):
            pltpu.sync_copy(acc.at[pl.ds(base + k * _CH, _CH)],
                            out_hbm.at[pl.ds(obase + k * _CH, _CH)])
        if rem:
            pltpu.sync_copy(acc.at[pl.ds(base + full * _CH, rem)],
                            out_hbm.at[pl.ds(obase + full * _CH, rem)])
        if tail:
            @pl.when(s == _NS - 1)
            def _():
                pltpu.sync_copy(acc.at[pl.ds(_NS * rpt, tail)],
                                out_hbm.at[pl.ds(coff + _NS * rpt, tail)])

    return run(src, dst, w, xt)


def _post(y1, y2, y3, n, d):
    hd = d // 2
    blk = 1000
    g = n // blk

    def body(y1_ref, y2_ref, y3_ref, o_ref, acc):
        p = pl.program_id(0)
        i = pl.program_id(1)
        s0 = y1_ref[0] + y2_ref[0] + y3_ref[0]
        s1 = y1_ref[1] + y2_ref[1] + y3_ref[1]

        @pl.when(jnp.logical_and(p == 0, i == 0))
        def _():
            acc[...] = jnp.zeros_like(acc)

        @pl.when(p == 0)
        def _():
            acc[0:1, :] += jnp.sum(s0, axis=0, keepdims=True)
            acc[1:2, :] += jnp.sum(s1, axis=0, keepdims=True)

        @pl.when(p == 1)
        def _():
            u0 = s0 - acc[0:1, :] / n
            u1 = s1 - acc[1:2, :] / n
            n2 = (jnp.sum(u0 * u0, axis=1, keepdims=True)
                  + jnp.sum(u1 * u1, axis=1, keepdims=True))
            un = jnp.maximum(jnp.sqrt(n2), _EPS)
            f = jnp.tanh(un) / un
            e0 = f * u0
            e1 = f * u1
            en2 = (jnp.sum(e0 * e0, axis=1, keepdims=True)
                   + jnp.sum(e1 * e1, axis=1, keepdims=True))
            en = jnp.maximum(jnp.sqrt(en2), _EPS)
            maxnorm = 1.0 - 4e-3
            scale = jnp.where(en > maxnorm, maxnorm / en, 1.0)
            o_ref[:, :hd] = e0 * scale
            o_ref[:, hd:] = e1 * scale

    return pl.pallas_call(
        body,
        grid=(2, g),
        in_specs=[pl.BlockSpec((2, blk, hd), lambda p, i: (0, i, 0))] * 3,
        out_specs=pl.BlockSpec((blk, d), lambda p, i: (i, 0)),
        out_shape=jax.ShapeDtypeStruct((n, d), jnp.float32),
        scratch_shapes=[pltpu.VMEM((2, hd), jnp.float32)],
    )(y1, y2, y3)


def kernel(x, edge_index, edge_weight):
    n, d = x.shape
    hd = d // 2
    e = edge_index.shape[1]

    ept = -(-e // (_NS * _CH)) * _CH
    e_pad = ept * _NS
    src = jnp.zeros((e_pad,), jnp.int32).at[:e].set(edge_index[1].astype(jnp.int32))
    dst = jnp.zeros((e_pad,), jnp.int32).at[:e].set(edge_index[0].astype(jnp.int32))
    w = jnp.zeros((e_pad,), jnp.float32).at[:e].set(edge_weight)

    xt = _pre_logmap(x, hd)
    y1 = _spmm_sc(src, dst, w, xt, n, hd)
    y2 = _spmm_sc(src, dst, w, y1, n, hd)
    y3 = _spmm_sc(src, dst, w, y2, n, hd)
    return _post(y1.reshape(2, n, hd), y2.reshape(2, n, hd),
                 y3.reshape(2, n, hd), n, d)


# speedup vs baseline: 3.2909x; 1.3252x over previous
import dataclasses
import functools

import jax
import jax.numpy as jnp
from jax import lax
from jax.experimental import pallas as pl
from jax.experimental.pallas import tpu as pltpu
from jax.experimental.pallas import tpu_sc as plsc

_EPS = 1e-15
_NUM_LAYERS = 3
_NC = 2
_NS = 16
_L = 16
_CH = 64
_G8 = 8
_WB = 64


def _pre_logmap(x, hd):
    n, d = x.shape
    blk = 1000

    def body(x_ref, o_ref):
        xb = x_ref[...]
        pn = jnp.sqrt(jnp.sum(xb * xb, axis=1, keepdims=True))
        pn = jnp.maximum(pn, _EPS)
        z = jnp.clip(pn, -1.0 + 1e-7, 1.0 - 1e-7)
        at = 0.5 * (jnp.log1p(z) - jnp.log1p(-z))
        xt = xb / pn * at
        o_ref[0] = xt[:, :hd]
        o_ref[1] = xt[:, hd:]

    out = pl.pallas_call(
        body,
        grid=(n // blk,),
        in_specs=[pl.BlockSpec((blk, d), lambda i: (i, 0))],
        out_specs=pl.BlockSpec((2, blk, hd), lambda i: (0, i, 0)),
        out_shape=jax.ShapeDtypeStruct((2, n, hd), jnp.float32),
    )(x)
    return out.reshape(2 * n, hd)


def _spmm_sc(pk, xt, n, hd, nch):
    ngrp = nch // _G8
    rpt = (n // _NS) // 8 * 8
    tail = n - _NS * rpt
    wfull = rpt // _WB
    wrem = rpt - wfull * _WB

    mesh = plsc.VectorSubcoreMesh(core_axis_name="c", subcore_axis_name="s")
    cp = pltpu.CompilerParams()
    if "needs_layout_passes" in pltpu.CompilerParams.__dataclass_fields__:
        cp = dataclasses.replace(cp, needs_layout_passes=False)

    @functools.partial(
        pl.kernel,
        out_type=jax.ShapeDtypeStruct((2 * n, hd), jnp.float32),
        mesh=mesh,
        compiler_params=cp,
        scratch_types=[
            pltpu.VMEM_SHARED((n, hd), jnp.float32),
            pltpu.VMEM((2, _G8, 4, _CH), jnp.int32),
            pltpu.VMEM((4, _CH, hd), jnp.float32),
            pltpu.SemaphoreType.DMA((4,)),
            pltpu.SemaphoreType.DMA((4,)),
            pltpu.SemaphoreType.DMA((2,)),
        ],
    )
    def run(pk_hbm, x_hbm, out_hbm, acc, ibuf, rows, gsem, ssem, isem):
        c = lax.axis_index("c")
        s = lax.axis_index("s")
        base = s * rpt

        @pl.loop(0, _WB)
        def _(i):
            for j in range(hd // _L):
                rows[0, i, pl.ds(j * _L, _L)] = jnp.zeros((_L,), jnp.float32)

        for k in range(wfull):
            pltpu.sync_copy(rows.at[0], acc.at[pl.ds(base + k * _WB, _WB)])
        if wrem:
            pltpu.sync_copy(rows.at[0, pl.ds(0, wrem)],
                            acc.at[pl.ds(base + wfull * _WB, wrem)])
        if tail:
            @pl.when(s == _NS - 1)
            def _():
                pltpu.sync_copy(rows.at[0, pl.ds(0, tail)],
                                acc.at[pl.ds(_NS * rpt, tail)])
        plsc.subcore_barrier()

        def idx_prefetch(q, g):
            pltpu.async_copy(pk_hbm.at[s, g], ibuf.at[q], isem.at[q])

        def wait_idx(q):
            pltpu.make_async_copy(pk_hbm.at[s, 0], ibuf.at[q],
                                  isem.at[q]).wait()

        def issue_gather(q, cc, b):
            pltpu.async_copy(x_hbm.at[ibuf.at[q, cc, c]], rows.at[b],
                             gsem.at[b])

        def wait_gather(q, cc, b):
            pltpu.make_async_copy(x_hbm.at[ibuf.at[q, cc, c]], rows.at[b],
                                  gsem.at[b]).wait()

        def wait_scatter(q, cc, b):
            pltpu.make_async_copy(rows.at[b], acc.at[ibuf.at[q, cc, 2]],
                                  ssem.at[b]).wait()

        def group_body(q, gnext, first=False, last=False):
            for cc in range(_G8):
                b = cc % 4
                if not (last and cc >= _G8 - 2):
                    if cc < _G8 - 2:
                        tq, tcc = q, cc + 2
                    else:
                        tq, tcc = 1 - q, cc + 2 - _G8
                    tb = (cc + 2) % 4
                    if not (first and cc < 2):
                        wait_scatter(tb)
                    if cc == 3 and not last:
                        idx_prefetch(1 - q, gnext)
                    if tq != q and cc == _G8 - 2:
                        wait_idx(1 - q)
                    issue_gather(tq, tcc, tb)
                wait_gather(b)

                @pl.loop(0, _CH, step=2)
                def _(e):
                    for u in range(2):
                        wv = plsc.bitcast(
                            plsc.load_gather(
                                ibuf.at[q, cc, 3],
                                [jnp.full((_L,), e + u, jnp.int32)]),
                            jnp.float32)
                        for j in range(hd // _L):
                            sl = pl.ds(j * _L, _L)
                            rows[b, e + u, sl] = rows[b, e + u, sl] * wv

                pltpu.async_copy(rows.at[b], acc.at[ibuf.at[q, cc, 2]],
                                 ssem.at[b], add=True)

        def group_body(q, gnext, first=False, last=False):
            for cc in range(_G8):
                b = cc % 4
                if not (last and cc >= _G8 - 2):
                    if cc < _G8 - 2:
                        tq, tcc = q, cc + 2
                    else:
                        tq, tcc = 1 - q, cc + 2 - _G8
                    tb = (cc + 2) % 4
                    if not (first and cc < 2):
                        pq, pcc = (q, cc - 2) if cc >= 2 else (1 - q, cc + 6)
                        wait_scatter(pq, pcc, tb)
                    if cc == 3 and not last:
                        idx_prefetch(1 - q, gnext)
                    if tq != q and cc == _G8 - 2:
                        wait_idx(1 - q)
                    issue_gather(tq, tcc, tb)
                wait_gather(q, cc, b)

                @pl.loop(0, _CH, step=2)
                def _(e):
                    for u in range(2):
                        wv = plsc.bitcast(
                            plsc.load_gather(
                                ibuf.at[q, cc, 3],
                                [jnp.full((_L,), e + u, jnp.int32)]),
                            jnp.float32)
                        for j in range(hd // _L):
                            sl = pl.ds(j * _L, _L)
                            rows[b, e + u, sl] = rows[b, e + u, sl] * wv

                pltpu.async_copy(rows.at[b], acc.at[ibuf.at[q, cc, 2]],
                                 ssem.at[b], add=True)

        pltpu.sync_copy(pk_hbm.at[s, 0], ibuf.at[0])
        issue_gather(0, 0, 0)
        issue_gather(0, 1, 1)
        group_body(0, 1, first=True)

        @pl.loop(0, (ngrp - 2) // 2)
        def _(i):
            ga = 1 + 2 * i
            group_body(1, ga + 1)
            group_body(0, ga + 2)

        group_body(1, 0, last=True)
        for cc in range(4, _G8):
            wait_scatter(1, cc, cc % 4)

        plsc.subcore_barrier()

        obase = c * n + base
        for k in range(wfull):
            pltpu.sync_copy(acc.at[pl.ds(base + k * _WB, _WB)],
                            out_hbm.at[pl.ds(obase + k * _WB, _WB)])
        if wrem:
            pltpu.sync_copy(acc.at[pl.ds(base + wfull * _WB, wrem)],
                            out_hbm.at[pl.ds(obase + wfull * _WB, wrem)])
        if tail:
            @pl.when(s == _NS - 1)
            def _():
                pltpu.sync_copy(acc.at[pl.ds(_NS * rpt, tail)],
                                out_hbm.at[pl.ds(c * n + _NS * rpt, tail)])

    return run(pk, xt)


def _post(y1, y2, y3, n, d):
    hd = d // 2
    blk = 1000
    g = n // blk

    def body(y1_ref, y2_ref, y3_ref, o_ref, acc):
        p = pl.program_id(0)
        i = pl.program_id(1)
        s0 = y1_ref[0] + y2_ref[0] + y3_ref[0]
        s1 = y1_ref[1] + y2_ref[1] + y3_ref[1]

        @pl.when(jnp.logical_and(p == 0, i == 0))
        def _():
            acc[...] = jnp.zeros_like(acc)

        @pl.when(p == 0)
        def _():
            acc[0:1, :] += jnp.sum(s0, axis=0, keepdims=True)
            acc[1:2, :] += jnp.sum(s1, axis=0, keepdims=True)

        @pl.when(p == 1)
        def _():
            u0 = s0 - acc[0:1, :] / n
            u1 = s1 - acc[1:2, :] / n
            n2 = (jnp.sum(u0 * u0, axis=1, keepdims=True)
                  + jnp.sum(u1 * u1, axis=1, keepdims=True))
            un = jnp.maximum(jnp.sqrt(n2), _EPS)
            f = jnp.tanh(un) / un
            e0 = f * u0
            e1 = f * u1
            en2 = (jnp.sum(e0 * e0, axis=1, keepdims=True)
                   + jnp.sum(e1 * e1, axis=1, keepdims=True))
            en = jnp.maximum(jnp.sqrt(en2), _EPS)
            maxnorm = 1.0 - 4e-3
            scale = jnp.where(en > maxnorm, maxnorm / en, 1.0)
            o_ref[:, :hd] = e0 * scale
            o_ref[:, hd:] = e1 * scale

    return pl.pallas_call(
        body,
        grid=(2, g),
        in_specs=[pl.BlockSpec((2, blk, hd), lambda p, i: (0, i, 0))] * 3,
        out_specs=pl.BlockSpec((blk, d), lambda p, i: (i, 0)),
        out_shape=jax.ShapeDtypeStruct((n, d), jnp.float32),
        scratch_shapes=[pltpu.VMEM((2, hd), jnp.float32)],
    )(y1, y2, y3)


def kernel(x, edge_index, edge_weight):
    n, d = x.shape
    hd = d // 2
    e = edge_index.shape[1]

    nch = -(-e // (_NS * _CH * 2 * _G8)) * 2 * _G8
    ept = nch * _CH
    e_pad = ept * _NS
    src = jnp.zeros((e_pad,), jnp.int32).at[:e].set(edge_index[1].astype(jnp.int32))
    dst = jnp.zeros((e_pad,), jnp.int32).at[:e].set(edge_index[0].astype(jnp.int32))
    w32 = lax.bitcast_convert_type(
        jnp.zeros((e_pad,), jnp.float32).at[:e].set(edge_weight), jnp.int32)
    pk = jnp.stack([src, src + n, dst, w32], axis=0)
    pk = pk.reshape(4, _NS, nch // _G8, _G8, _CH).transpose(1, 2, 3, 0, 4)

    xt = _pre_logmap(x, hd)
    y1 = _spmm_sc(pk, xt, n, hd, nch)
    y2 = _spmm_sc(pk, y1, n, hd, nch)
    y3 = _spmm_sc(pk, y2, n, hd, nch)
    return _post(y1.reshape(2, n, hd), y2.reshape(2, n, hd),
                 y3.reshape(2, n, hd), n, d)
